# baseline (device time: 156504 ns/iter reference)
import jax
import jax.numpy as jnp
from jax import lax
from jax.experimental import pallas as pl
from jax.experimental.pallas import tpu as pltpu

M = 3072
K_SHARD = 1536
N = 3072
M_HALF = M // 2
BM = 288
BM_LAST = 96
NBLK = 6


def kernel(A, B):
    def body(
        a_ref, b_ref, out_ref,
        send_buf, recv_buf, sum_buf, b16_ref,
        x_send_sems, x_recv_sems,
        y_send_sems, y_recv_sems,
        copy_sems, credit_sem,
    ):
        i = pl.program_id(0)
        my_x = lax.axis_index("x")
        my_y = lax.axis_index("y")
        xpeer = (1 - my_x, my_y)
        ypeer = (my_x, 1 - my_y)

        slot = i % 2
        pslot = (i - 1) % 2
        row = my_y * M_HALF + i * BM
        prow = row - BM

        def x_rdma_at(s, bm=BM):
            return pltpu.make_async_remote_copy(
                src_ref=send_buf.at[s, pl.ds(0, bm)],
                dst_ref=recv_buf.at[s, pl.ds(0, bm)],
                send_sem=x_send_sems.at[s],
                recv_sem=x_recv_sems.at[s],
                device_id=xpeer,
                device_id_type=pl.DeviceIdType.MESH,
            )

        def y_rdma_at(s, sem, r, bm=BM):
            return pltpu.make_async_remote_copy(
                src_ref=sum_buf.at[s, pl.ds(0, bm)],
                dst_ref=out_ref.at[pl.ds(r, bm), :],
                send_sem=y_send_sems.at[sem],
                recv_sem=y_recv_sems.at[sem],
                device_id=ypeer,
                device_id_type=pl.DeviceIdType.MESH,
            )

        def local_copy_at(s, r, bm=BM):
            return pltpu.make_async_copy(
                sum_buf.at[s, pl.ds(0, bm)],
                out_ref.at[pl.ds(r, bm), :],
                copy_sems.at[s],
            )

        def partial(r, bm):
            return jnp.dot(
                a_ref[pl.ds(r, bm), :].astype(jnp.bfloat16), b16_ref[...],
                preferred_element_type=jnp.float32,
            ).astype(jnp.bfloat16)

        @pl.when(i == 0)
        def _():
            barrier = pltpu.get_barrier_semaphore()
            for nbr in (xpeer, ypeer):
                pl.semaphore_signal(
                    barrier, inc=1, device_id=nbr,
                    device_id_type=pl.DeviceIdType.MESH,
                )
            pl.semaphore_wait(barrier, 2)
            b16_ref[...] = b_ref[...].astype(jnp.bfloat16)
            send_buf[0, pl.ds(0, BM)] = partial(row, BM)
            x_rdma_at(0).start()

        @pl.when(i + 1 < NBLK)
        def _():
            @pl.when(i >= 1)
            def _():
                x_rdma_at(pslot).wait_send()

            @pl.when(i + 1 < NBLK - 1)
            def _():
                send_buf[pslot, pl.ds(0, BM)] = partial(row + BM, BM)

            @pl.when(i + 1 == NBLK - 1)
            def _():
                send_buf[pslot, pl.ds(0, BM_LAST)] = partial(row + BM, BM_LAST)

            @pl.when(i >= 1)
            def _():
                pl.semaphore_wait(credit_sem, 1)

            @pl.when(i + 1 < NBLK - 1)
            def _():
                x_rdma_at(pslot).start()

            @pl.when(i + 1 == NBLK - 1)
            def _():
                x_rdma_at(pslot, BM_LAST).start()

        @pl.when(i < NBLK - 1)
        def _():
            x_rdma_at(slot).wait_recv()
            sum_buf[slot, pl.ds(0, BM)] = (
                recv_buf[slot, pl.ds(0, BM)] + send_buf[slot, pl.ds(0, BM)]
            )

        @pl.when(i == NBLK - 1)
        def _():
            x_rdma_at(slot, BM_LAST).wait_recv()
            sum_buf[slot, pl.ds(0, BM_LAST)] = (
                recv_buf[slot, pl.ds(0, BM_LAST)]
                + send_buf[slot, pl.ds(0, BM_LAST)]
            )

        @pl.when(i <= NBLK - 3)
        def _():
            pl.semaphore_signal(
                credit_sem, inc=1, device_id=xpeer,
                device_id_type=pl.DeviceIdType.MESH,
            )

        @pl.when(i < NBLK - 1)
        def _():
            y_rdma_at(slot, i, row).start()
            local_copy_at(slot, row).start()

        @pl.when(i >= 1)
        def _():
            y_prev = y_rdma_at(pslot, i - 1, prow)
            y_prev.wait_send()
            y_prev.wait_recv()
            local_copy_at(pslot, prow).wait()

        @pl.when(i == NBLK - 1)
        def _():
            y_last = y_rdma_at(slot, i, row, BM_LAST)
            y_last.start()
            local = local_copy_at(slot, row, BM_LAST)
            local.start()
            y_last.wait_send()
            y_last.wait_recv()
            local.wait()
            x_rdma_at(slot, BM_LAST).wait_send()
            x_rdma_at(pslot).wait_send()

    return pl.pallas_call(
        body,
        grid=(NBLK,),
        out_shape=jax.ShapeDtypeStruct((M, N), jnp.bfloat16),
        in_specs=[
            pl.BlockSpec(memory_space=pltpu.VMEM),
            pl.BlockSpec(memory_space=pltpu.VMEM),
        ],
        out_specs=pl.BlockSpec(memory_space=pl.ANY),
        scratch_shapes=[
            pltpu.VMEM((2, BM, N), jnp.bfloat16),
            pltpu.VMEM((2, BM, N), jnp.bfloat16),
            pltpu.VMEM((2, BM, N), jnp.bfloat16),
            pltpu.VMEM((K_SHARD, N), jnp.bfloat16),
            pltpu.SemaphoreType.DMA((2,)),
            pltpu.SemaphoreType.DMA((2,)),
            pltpu.SemaphoreType.DMA((NBLK,)),
            pltpu.SemaphoreType.DMA((NBLK,)),
            pltpu.SemaphoreType.DMA((2,)),
            pltpu.SemaphoreType.REGULAR,
        ],
        compiler_params=pltpu.CompilerParams(
            collective_id=0,
            dimension_semantics=("arbitrary",),
            vmem_limit_bytes=60 * 1024 * 1024,
        ),
    )(A, B)


# device time: 147273 ns/iter; 1.0627x vs baseline; 1.0627x over previous
import jax
import jax.numpy as jnp
from jax import lax
from jax.experimental import pallas as pl
from jax.experimental.pallas import tpu as pltpu

M = 3072
K_SHARD = 1536
N = 3072
BM = 192
NBLK = M // (2 * BM)


def kernel(A, B):
    def body(
        a_ref, b_ref, out_ref,
        send_buf, recv_buf, sum_buf, b16_ref,
        x_send_sems, x_recv_sems,
        y_send_sems, y_recv_sems,
        copy_sems, credit_sem,
    ):
        i = pl.program_id(0)
        my_x = lax.axis_index("x")
        my_y = lax.axis_index("y")
        xpeer = (1 - my_x, my_y)
        ypeer = (my_x, 1 - my_y)

        slot = i % 2
        pslot = (i - 1) % 2
        row = (my_y * NBLK + i) * BM
        prow = (my_y * NBLK + i - 1) * BM

        def x_rdma_at(s):
            return pltpu.make_async_remote_copy(
                src_ref=send_buf.at[s],
                dst_ref=recv_buf.at[s],
                send_sem=x_send_sems.at[s],
                recv_sem=x_recv_sems.at[s],
                device_id=xpeer,
                device_id_type=pl.DeviceIdType.MESH,
            )

        def y_rdma_at(s, sem, r):
            return pltpu.make_async_remote_copy(
                src_ref=sum_buf.at[s],
                dst_ref=out_ref.at[pl.ds(r, BM), :],
                send_sem=y_send_sems.at[sem],
                recv_sem=y_recv_sems.at[sem],
                device_id=ypeer,
                device_id_type=pl.DeviceIdType.MESH,
            )

        def local_copy_at(s, r):
            return pltpu.make_async_copy(
                sum_buf.at[s], out_ref.at[pl.ds(r, BM), :], copy_sems.at[s]
            )

        @pl.when(i == 0)
        def _():
            barrier = pltpu.get_barrier_semaphore()
            for nbr in (xpeer, ypeer):
                pl.semaphore_signal(
                    barrier, inc=1, device_id=nbr,
                    device_id_type=pl.DeviceIdType.MESH,
                )
            pl.semaphore_wait(barrier, 2)
            b16_ref[...] = b_ref[...].astype(jnp.bfloat16)
            send_buf[0] = jnp.dot(
                a_ref[pl.ds(row, BM), :].astype(jnp.bfloat16), b16_ref[...],
                preferred_element_type=jnp.float32,
            ).astype(jnp.bfloat16)
            x_rdma_at(0).start()

        @pl.when(i + 1 < NBLK)
        def _():
            @pl.when(i >= 1)
            def _():
                x_rdma_at(pslot).wait_send()
            send_buf[pslot] = jnp.dot(
                a_ref[pl.ds(row + BM, BM), :].astype(jnp.bfloat16),
                b16_ref[...],
                preferred_element_type=jnp.float32,
            ).astype(jnp.bfloat16)

            @pl.when(i >= 1)
            def _():
                pl.semaphore_wait(credit_sem, 1)

            x_rdma_at(pslot).start()

        x_rdma_at(slot).wait_recv()
        sum_buf[slot] = recv_buf[slot] + send_buf[slot]

        @pl.when(i <= NBLK - 3)
        def _():
            pl.semaphore_signal(
                credit_sem, inc=1, device_id=xpeer,
                device_id_type=pl.DeviceIdType.MESH,
            )

        y_rdma = y_rdma_at(slot, i, row)
        y_rdma.start()
        local = local_copy_at(slot, row)
        local.start()

        @pl.when(i >= 1)
        def _():
            y_prev = y_rdma_at(pslot, i - 1, prow)
            y_prev.wait_send()
            y_prev.wait_recv()
            local_copy_at(pslot, prow).wait()

        @pl.when(i == NBLK - 1)
        def _():
            y_rdma.wait_send()
            y_rdma.wait_recv()
            local.wait()
            x_rdma_at(slot).wait_send()
            x_rdma_at(pslot).wait_send()

    return pl.pallas_call(
        body,
        grid=(NBLK,),
        out_shape=jax.ShapeDtypeStruct((M, N), jnp.bfloat16),
        in_specs=[
            pl.BlockSpec(memory_space=pltpu.VMEM),
            pl.BlockSpec(memory_space=pltpu.VMEM),
        ],
        out_specs=pl.BlockSpec(memory_space=pl.ANY),
        scratch_shapes=[
            pltpu.VMEM((2, BM, N), jnp.bfloat16),
            pltpu.VMEM((2, BM, N), jnp.bfloat16),
            pltpu.VMEM((2, BM, N), jnp.bfloat16),
            pltpu.VMEM((K_SHARD, N), jnp.bfloat16),
            pltpu.SemaphoreType.DMA((2,)),
            pltpu.SemaphoreType.DMA((2,)),
            pltpu.SemaphoreType.DMA((NBLK,)),
            pltpu.SemaphoreType.DMA((NBLK,)),
            pltpu.SemaphoreType.DMA((2,)),
            pltpu.SemaphoreType.REGULAR,
        ],
        compiler_params=pltpu.CompilerParams(
            collective_id=0,
            dimension_semantics=("arbitrary",),
            vmem_limit_bytes=60 * 1024 * 1024,
        ),
    )(A, B)


# device time: 141233 ns/iter; 1.1081x vs baseline; 1.0428x over previous
import jax
import jax.numpy as jnp
from jax import lax
from jax.experimental import pallas as pl
from jax.experimental.pallas import tpu as pltpu

M = 3072
K_SHARD = 1536
N = 3072
BM = 96
NBLK = M // (2 * BM)


def kernel(A, B):
    def body(
        a_ref, b_ref, out_ref,
        send_buf, recv_buf, sum_buf, b16_ref,
        x_send_sems, x_recv_sems,
        y_send_sems, y_recv_sems,
        copy_sems, credit_sem,
    ):
        i = pl.program_id(0)
        my_x = lax.axis_index("x")
        my_y = lax.axis_index("y")
        xpeer = (1 - my_x, my_y)
        ypeer = (my_x, 1 - my_y)

        slot = i % 2
        pslot = (i - 1) % 2
        row = (my_y * NBLK + i) * BM
        prow = (my_y * NBLK + i - 1) * BM

        def x_rdma_at(s):
            return pltpu.make_async_remote_copy(
                src_ref=send_buf.at[s],
                dst_ref=recv_buf.at[s],
                send_sem=x_send_sems.at[s],
                recv_sem=x_recv_sems.at[s],
                device_id=xpeer,
                device_id_type=pl.DeviceIdType.MESH,
            )

        def y_rdma_at(s, sem, r):
            return pltpu.make_async_remote_copy(
                src_ref=sum_buf.at[s],
                dst_ref=out_ref.at[pl.ds(r, BM), :],
                send_sem=y_send_sems.at[sem],
                recv_sem=y_recv_sems.at[sem],
                device_id=ypeer,
                device_id_type=pl.DeviceIdType.MESH,
            )

        def local_copy_at(s, r):
            return pltpu.make_async_copy(
                sum_buf.at[s], out_ref.at[pl.ds(r, BM), :], copy_sems.at[s]
            )

        @pl.when(i == 0)
        def _():
            barrier = pltpu.get_barrier_semaphore()
            for nbr in (xpeer, ypeer):
                pl.semaphore_signal(
                    barrier, inc=1, device_id=nbr,
                    device_id_type=pl.DeviceIdType.MESH,
                )
            pl.semaphore_wait(barrier, 2)
            b16_ref[...] = b_ref[...].astype(jnp.bfloat16)
            send_buf[0] = jnp.dot(
                a_ref[pl.ds(row, BM), :].astype(jnp.bfloat16), b16_ref[...],
                preferred_element_type=jnp.float32,
            ).astype(jnp.bfloat16)
            x_rdma_at(0).start()

        @pl.when(i + 1 < NBLK)
        def _():
            @pl.when(i >= 1)
            def _():
                x_rdma_at(pslot).wait_send()
            send_buf[pslot] = jnp.dot(
                a_ref[pl.ds(row + BM, BM), :].astype(jnp.bfloat16),
                b16_ref[...],
                preferred_element_type=jnp.float32,
            ).astype(jnp.bfloat16)

            @pl.when(i >= 1)
            def _():
                pl.semaphore_wait(credit_sem, 1)

            x_rdma_at(pslot).start()

        x_rdma_at(slot).wait_recv()
        sum_buf[slot] = recv_buf[slot] + send_buf[slot]

        @pl.when(i <= NBLK - 3)
        def _():
            pl.semaphore_signal(
                credit_sem, inc=1, device_id=xpeer,
                device_id_type=pl.DeviceIdType.MESH,
            )

        y_rdma = y_rdma_at(slot, i, row)
        y_rdma.start()
        local = local_copy_at(slot, row)
        local.start()

        @pl.when(i >= 1)
        def _():
            y_prev = y_rdma_at(pslot, i - 1, prow)
            y_prev.wait_send()
            y_prev.wait_recv()
            local_copy_at(pslot, prow).wait()

        @pl.when(i == NBLK - 1)
        def _():
            y_rdma.wait_send()
            y_rdma.wait_recv()
            local.wait()
            x_rdma_at(slot).wait_send()
            x_rdma_at(pslot).wait_send()

    return pl.pallas_call(
        body,
        grid=(NBLK,),
        out_shape=jax.ShapeDtypeStruct((M, N), jnp.bfloat16),
        in_specs=[
            pl.BlockSpec(memory_space=pltpu.VMEM),
            pl.BlockSpec(memory_space=pltpu.VMEM),
        ],
        out_specs=pl.BlockSpec(memory_space=pl.ANY),
        scratch_shapes=[
            pltpu.VMEM((2, BM, N), jnp.bfloat16),
            pltpu.VMEM((2, BM, N), jnp.bfloat16),
            pltpu.VMEM((2, BM, N), jnp.bfloat16),
            pltpu.VMEM((K_SHARD, N), jnp.bfloat16),
            pltpu.SemaphoreType.DMA((2,)),
            pltpu.SemaphoreType.DMA((2,)),
            pltpu.SemaphoreType.DMA((NBLK,)),
            pltpu.SemaphoreType.DMA((NBLK,)),
            pltpu.SemaphoreType.DMA((2,)),
            pltpu.SemaphoreType.REGULAR,
        ],
        compiler_params=pltpu.CompilerParams(
            collective_id=0,
            dimension_semantics=("arbitrary",),
            vmem_limit_bytes=60 * 1024 * 1024,
        ),
    )(A, B)
